# baseline (speedup 1.0000x reference)
import jax
import jax.numpy as jnp
from jax import lax
from jax.experimental import pallas as pl

_H = 256
_W = 256
_K = 18
_BH = 256


def _box_kernel(rlo_ref, clo_ref, out_ref):
    wk = _W * _K
    lane = lax.broadcasted_iota(jnp.int32, (1, wk), 1)
    j_id = lane // _K
    cd = (j_id - clo_ref[0]).astype(jnp.uint32)
    colf = jnp.where(cd <= 10, jnp.float32(1.0), jnp.float32(0.0))
    base = pl.program_id(1) * _BH
    ri = base + lax.broadcasted_iota(jnp.int32, (_BH, wk), 0)
    rd = (ri - rlo_ref[0]).astype(jnp.uint32)
    out_ref[0] = jnp.where(rd <= 10, colf, jnp.float32(0.0))


def _half(rlo, clo, b, k):
    wk = _W * _K
    out = pl.pallas_call(
        _box_kernel,
        grid=(b, _H // _BH),
        in_specs=[
            pl.BlockSpec((1, 1, wk), lambda bi, hi: (bi, 0, 0)),
            pl.BlockSpec((1, 1, wk), lambda bi, hi: (bi, 0, 0)),
        ],
        out_specs=pl.BlockSpec((1, _BH, wk), lambda bi, hi: (bi, hi, 0)),
        out_shape=jax.ShapeDtypeStruct((b, _H, wk), jnp.float32),
    )(rlo, clo)
    return out.reshape(b, _H, _W, k)


def kernel(x):
    b, k, _ = x.shape
    wk = _W * _K
    rlo = jnp.broadcast_to((x[:, :, 0] - 5)[:, None, :], (b, _W, k)).reshape(b, 1, wk)
    clo = jnp.broadcast_to((x[:, :, 1] - 5)[:, None, :], (b, _W, k)).reshape(b, 1, wk)
    h = b // 2
    out0 = _half(rlo[:h], clo[:h], h, k)
    out1 = _half(rlo[h:], clo[h:], h, k)
    return jnp.concatenate([out0, out1], axis=0)


# speedup vs baseline: 2.0306x; 2.0306x over previous
import jax
import jax.numpy as jnp
from jax import lax
from jax.experimental import pallas as pl
from jax.experimental.pallas import tpu as pltpu

_H = 256
_W = 256
_K = 18


def _box_kernel(xr_ref, xc_ref, out_ref):
    bi = pl.program_id(0)
    ki = pl.program_id(1)
    r0 = xr_ref[bi, ki]
    c0 = xc_ref[bi, ki]
    ri = lax.broadcasted_iota(jnp.int32, (_H, 1), 0)
    rowf = jnp.where((ri - r0).astype(jnp.uint32) <= 10,
                     jnp.float32(1.0), jnp.float32(0.0))
    cj = lax.broadcasted_iota(jnp.int32, (1, _W), 1)
    colf = jnp.where((cj - c0).astype(jnp.uint32) <= 10,
                     jnp.float32(1.0), jnp.float32(0.0))
    out_ref[0, 0] = rowf * colf


def kernel(x):
    b, k, _ = x.shape
    xr = x[:, :, 0] - 5
    xc = x[:, :, 1] - 5
    grid_spec = pltpu.PrefetchScalarGridSpec(
        num_scalar_prefetch=2,
        grid=(b, k),
        in_specs=[],
        out_specs=pl.BlockSpec((1, 1, _H, _W),
                               lambda bi, ki, xr_s, xc_s: (bi, ki, 0, 0)),
    )
    y = pl.pallas_call(
        _box_kernel,
        grid_spec=grid_spec,
        out_shape=jax.ShapeDtypeStruct((b, k, _H, _W), jnp.float32),
    )(xr, xc)
    return jnp.transpose(y, (0, 2, 3, 1))


# speedup vs baseline: 8.3592x; 4.1165x over previous
import jax
import jax.numpy as jnp
from jax import lax
from jax.experimental import pallas as pl
from jax.experimental.pallas import tpu as pltpu

_H = 256
_W = 256
_K = 18


def _box_kernel(xr_ref, xc_ref, out_ref):
    bi = pl.program_id(0)
    ri = lax.broadcasted_iota(jnp.int32, (_H, 1), 0)
    cj = lax.broadcasted_iota(jnp.int32, (1, _W), 1)
    for ki in range(_K):
        r0 = xr_ref[bi, ki]
        c0 = xc_ref[bi, ki]
        rowf = jnp.where((ri - r0).astype(jnp.uint32) <= 10,
                         jnp.float32(1.0), jnp.float32(0.0))
        colf = jnp.where((cj - c0).astype(jnp.uint32) <= 10,
                         jnp.float32(1.0), jnp.float32(0.0))
        out_ref[0, ki] = rowf * colf


def kernel(x):
    b, k, _ = x.shape
    xr = x[:, :, 0] - 5
    xc = x[:, :, 1] - 5
    grid_spec = pltpu.PrefetchScalarGridSpec(
        num_scalar_prefetch=2,
        grid=(b,),
        in_specs=[],
        out_specs=pl.BlockSpec((1, _K, _H, _W),
                               lambda bi, xr_s, xc_s: (bi, 0, 0, 0)),
    )
    y = pl.pallas_call(
        _box_kernel,
        grid_spec=grid_spec,
        out_shape=jax.ShapeDtypeStruct((b, k, _H, _W), jnp.float32),
    )(xr, xc)
    return jnp.transpose(y, (0, 2, 3, 1))
